# Initial kernel scaffold; baseline (speedup 1.0000x reference)
import jax
import jax.numpy as jnp
from jax.experimental import pallas as pl


def kernel(x, edge_index, Wl1, Wr1, att1, b1, Wl2, Wr2, att2, b2):
    raise NotImplementedError("write your pallas kernel here")


# speedup vs baseline: 6.0361x; 6.0361x over previous
import functools

import jax
import jax.numpy as jnp
import numpy as np
from jax import lax
from jax.experimental import pallas as pl
from jax.experimental.pallas import tpu as pltpu
from jax.experimental.pallas import tpu_sc as plsc

N = 10000
F = 128
C = 128
H1 = 4
N_PAD = 10240
SROWS = N_PAD // 128
SPAD = 128
E_REAL = 320000
E_SL = E_REAL + N
E_PAD = 331776
CHUNK = 64
RPT = N_PAD // 16
NBLK = SROWS // 16
EB = RPT // CHUNK

_f32 = jnp.float32
_i32 = jnp.int32


def _sc_edge_layer(n_heads):
    rounds = 2 if n_heads == 4 else 1
    span = E_PAD // 16 if n_heads == 4 else E_PAD // 32
    n_chunks = span // CHUNK

    mesh = plsc.VectorSubcoreMesh(core_axis_name="c", subcore_axis_name="s")
    if n_heads == 4:
        out_type = jax.ShapeDtypeStruct((H1 * N_PAD, C), _f32)
    else:
        out_type = (jax.ShapeDtypeStruct((2 * N_PAD, C), _f32),
                    jax.ShapeDtypeStruct((2 * SPAD, 128), _f32))

    @functools.partial(
        pl.kernel,
        out_type=out_type,
        mesh=mesh,
        scratch_types=[
            pltpu.VMEM((CHUNK,), _i32),
            pltpu.VMEM((CHUNK,), _i32),
            pltpu.VMEM((CHUNK,), _i32),
            pltpu.VMEM((CHUNK,), _i32),
            pltpu.VMEM((CHUNK,), _i32),
            pltpu.VMEM((CHUNK, C), _f32),
            pltpu.VMEM((CHUNK, C), _f32),
            pltpu.VMEM((CHUNK, C), _f32),
            pltpu.VMEM((CHUNK, 128), _f32),
            pltpu.VMEM((C,), _f32),
            pltpu.VMEM((16, 128), _f32),
            pltpu.VMEM_SHARED((N_PAD, C), _f32),
            pltpu.VMEM_SHARED((SPAD, 128), _f32),
            pltpu.SemaphoreType.DMA,
            pltpu.SemaphoreType.DMA,
        ],
    )
    def body(xl_hbm, xr_hbm, srch_hbm, dsth_hbm, dst_hbm, dstg_hbm, dstc_hbm,
             att_hbm, *rest):
        if n_heads == 4:
            (out_hbm, sv, dv, dav, sgv, colv, xl_v, xr_v, w_v, oh_v,
             att_v, s_loc, acc_sh, sden_sh, sem_l, sem_r) = rest
        else:
            (out_hbm, sout_hbm, sv, dv, dav, sgv, colv, xl_v, xr_v, w_v, oh_v,
             att_v, s_loc, acc_sh, sden_sh, sem_l, sem_r) = rest
        cid = lax.axis_index("c")
        sid = lax.axis_index("s")
        lane = lax.iota(_i32, 16)
        lanes = [lane + 16 * j for j in range(8)]
        shuf = [lane ^ sh for sh in (8, 4, 2, 1)]

        for r in range(rounds):
            if n_heads == 4:
                head = cid * 2 + r
                base0 = sid * span
            else:
                head = 0
                base0 = (cid * 16 + sid) * span

            @pl.loop(0, CHUNK)
            def _zero(e):
                for j in range(C // 16):
                    w_v[e, pl.ds(j * 16, 16)] = jnp.zeros((16,), _f32)

            for b in range(EB):
                row = sid * RPT + b * CHUNK
                pltpu.sync_copy(w_v, acc_sh.at[pl.ds(row, CHUNK)])
            pltpu.sync_copy(w_v.at[pl.ds(0, 8)],
                            sden_sh.at[pl.ds(sid * 8, 8)])

            att_off = pl.multiple_of(head * C, C)
            pltpu.sync_copy(att_hbm.at[pl.ds(att_off, C)], att_v)
            plsc.subcore_barrier()

            ebase0 = pl.multiple_of(head * E_PAD + base0, CHUNK)

            @pl.loop(0, n_chunks)
            def _edges(g):
                base = pl.multiple_of(base0 + g * CHUNK, CHUNK)
                ebase = pl.multiple_of(ebase0 + g * CHUNK, CHUNK)
                pltpu.sync_copy(srch_hbm.at[pl.ds(ebase, CHUNK)], sv)
                pltpu.sync_copy(dsth_hbm.at[pl.ds(ebase, CHUNK)], dav)
                pltpu.sync_copy(dst_hbm.at[pl.ds(base, CHUNK)], dv)
                pltpu.sync_copy(dstg_hbm.at[pl.ds(base, CHUNK)], sgv)
                pltpu.sync_copy(dstc_hbm.at[pl.ds(base, CHUNK)], colv)
                gl = pltpu.async_copy(xl_hbm.at[sv], xl_v, sem_l)
                gr = pltpu.async_copy(xr_hbm.at[dav], xr_v, sem_r)
                gl.wait()
                gr.wait()

                @pl.loop(0, CHUNK)
                def _edge(e):
                    accv = jnp.zeros((16,), _f32)
                    vls = []
                    for j in range(C // 16):
                        sl = pl.ds(j * 16, 16)
                        vl = xl_v[e, sl]
                        vls.append(vl)
                        v = vl + xr_v[e, sl]
                        lk = jnp.maximum(v, v * 0.2)
                        accv = accv + lk * att_v[sl]
                    for s in shuf:
                        accv = accv + jnp.take(accv, s)
                    p = jnp.exp(accv)
                    for j in range(C // 16):
                        w_v[e, pl.ds(j * 16, 16)] = p * vls[j]
                    cs = colv[pl.ds(pl.multiple_of((e >> 4) * 16, 16), 16)]
                    col = jnp.take(cs, jnp.full((16,), e & 15, _i32))
                    for j in range(8):
                        oh_v[e, pl.ds(j * 16, 16)] = jnp.where(
                            lanes[j] == col, p, 0.0)

                pltpu.sync_copy(w_v, acc_sh.at[dv], add=True)
                pltpu.sync_copy(oh_v, sden_sh.at[sgv], add=True)

            plsc.subcore_barrier()

            swin = pl.multiple_of(((sid * NBLK) >> 3) << 3, 8)
            pltpu.sync_copy(sden_sh.at[pl.ds(swin, 16)], s_loc)
            for b in range(EB):
                row = sid * RPT + b * CHUNK
                pltpu.sync_copy(acc_sh.at[pl.ds(row, CHUNK)], w_v)
                if n_heads == 4:
                    @pl.loop(0, CHUNK)
                    def _norm(e):
                        li = b * CHUNK + e
                        rw = sid * NBLK + (li >> 7) - swin
                        ss = s_loc[rw,
                                   pl.ds(pl.multiple_of(((li >> 4) & 7) * 16, 16), 16)]
                        den = jnp.take(ss, jnp.full((16,), li & 15, _i32))
                        inv = 1.0 / (den + 1e-16)
                        for j in range(C // 16):
                            sl = pl.ds(j * 16, 16)
                            w_v[e, sl] = w_v[e, sl] * inv

                    orow = pl.multiple_of(head * N_PAD + row, CHUNK)
                else:
                    orow = pl.multiple_of(cid * N_PAD + row, CHUNK)
                pltpu.sync_copy(w_v, out_hbm.at[pl.ds(orow, CHUNK)])
            if n_heads == 1:
                srow = pl.multiple_of(cid * SPAD + sid * 8, 8)
                pltpu.sync_copy(sden_sh.at[pl.ds(pl.multiple_of(sid * 8, 8), 8)],
                                sout_hbm.at[pl.ds(srow, 8)])
            if r + 1 < rounds:
                plsc.subcore_barrier()

    return body


def _sc_merge():
    mesh = plsc.VectorSubcoreMesh(core_axis_name="c", subcore_axis_name="s")

    @functools.partial(
        pl.kernel,
        out_type=jax.ShapeDtypeStruct((N_PAD, C), _f32),
        mesh=mesh,
        scratch_types=[
            pltpu.VMEM((CHUNK, C), _f32),
            pltpu.VMEM((CHUNK, C), _f32),
            pltpu.VMEM((16, 128), _f32),
            pltpu.VMEM((16, 128), _f32),
            pltpu.VMEM((C,), _f32),
        ],
    )
    def body(num_hbm, s_hbm, b2_hbm, out_hbm, a_v, b_v, s0_v, s1_v, b2_v):
        cid = lax.axis_index("c")
        sid = lax.axis_index("s")

        @pl.when(cid == 0)
        def _():
            pltpu.sync_copy(b2_hbm, b2_v)
            swin = pl.multiple_of(((sid * NBLK) >> 3) << 3, 8)
            pltpu.sync_copy(s_hbm.at[pl.ds(swin, 16)], s0_v)
            pltpu.sync_copy(s_hbm.at[pl.ds(SPAD + swin, 16)], s1_v)
            for b in range(EB):
                row = sid * RPT + b * CHUNK
                pltpu.sync_copy(num_hbm.at[pl.ds(row, CHUNK)], a_v)
                pltpu.sync_copy(num_hbm.at[pl.ds(N_PAD + row, CHUNK)], b_v)

                @pl.loop(0, CHUNK)
                def _m(e):
                    li = b * CHUNK + e
                    rw = sid * NBLK + (li >> 7) - swin
                    sl16 = pl.ds(pl.multiple_of(((li >> 4) & 7) * 16, 16), 16)
                    ss = s0_v[rw, sl16] + s1_v[rw, sl16]
                    den = jnp.take(ss, jnp.full((16,), li & 15, _i32))
                    inv = 1.0 / (den + 1e-16)
                    for j in range(C // 16):
                        sl = pl.ds(j * 16, 16)
                        a_v[e, sl] = (a_v[e, sl] + b_v[e, sl]) * inv + b2_v[sl]

                pltpu.sync_copy(a_v, out_hbm.at[pl.ds(row, CHUNK)])

    return body


def _tc_mm1(x_pad, Wl1, Wr1):
    TN = 1024
    nt = N_PAD // TN

    def body(x_ref, wl_ref, wr_ref, ol_ref, or_ref):
        x = x_ref[...]
        ol_ref[...] = jnp.dot(x, wl_ref[...], preferred_element_type=_f32)
        or_ref[...] = jnp.dot(x, wr_ref[...], preferred_element_type=_f32)

    return pl.pallas_call(
        body,
        grid=(nt, H1),
        in_specs=[
            pl.BlockSpec((TN, F), lambda n, h: (n, 0)),
            pl.BlockSpec((F, C), lambda n, h: (0, h)),
            pl.BlockSpec((F, C), lambda n, h: (0, h)),
        ],
        out_specs=[
            pl.BlockSpec((TN, C), lambda n, h: (h * nt + n, 0)),
            pl.BlockSpec((TN, C), lambda n, h: (h * nt + n, 0)),
        ],
        out_shape=[
            jax.ShapeDtypeStruct((H1 * N_PAD, C), _f32),
            jax.ShapeDtypeStruct((H1 * N_PAD, C), _f32),
        ],
    )(x_pad, Wl1, Wr1)


def _tc_mid(msg1, b1, Wl2, Wr2):
    TN = 1024
    inv_sqrt2 = 1.0 / np.sqrt(2.0)

    def body(msg_ref, b1_ref, wl_ref, wr_ref, ol_ref, or_ref):
        rows = pl.program_id(0) * TN + lax.broadcasted_iota(_i32, (TN, 1), 0)
        valid = rows < N
        accl = jnp.zeros((TN, C), _f32)
        accr = jnp.zeros((TN, C), _f32)
        for h in range(H1):
            v = msg_ref[h] + b1_ref[h][None, :]
            g = 0.5 * v * (1.0 + lax.erf(v * inv_sqrt2))
            g = jnp.where(valid, g, 0.0)
            accl = accl + jnp.dot(g, wl_ref[h * C:(h + 1) * C, :],
                                  preferred_element_type=_f32)
            accr = accr + jnp.dot(g, wr_ref[h * C:(h + 1) * C, :],
                                  preferred_element_type=_f32)
        ol_ref[...] = accl
        or_ref[...] = accr

    return pl.pallas_call(
        body,
        grid=(N_PAD // TN,),
        in_specs=[
            pl.BlockSpec((H1, TN, C), lambda n: (0, n, 0)),
            pl.BlockSpec((H1, C), lambda n: (0, 0)),
            pl.BlockSpec((H1 * C, C), lambda n: (0, 0)),
            pl.BlockSpec((H1 * C, C), lambda n: (0, 0)),
        ],
        out_specs=[
            pl.BlockSpec((TN, C), lambda n: (n, 0)),
            pl.BlockSpec((TN, C), lambda n: (n, 0)),
        ],
        out_shape=[
            jax.ShapeDtypeStruct((N_PAD, C), _f32),
            jax.ShapeDtypeStruct((N_PAD, C), _f32),
        ],
    )(msg1.reshape(H1, N_PAD, C), b1.reshape(H1, C), Wl2, Wr2)


def kernel(x, edge_index, Wl1, Wr1, att1, b1, Wl2, Wr2, att2, b2):
    loop_idx = jnp.arange(N, dtype=_i32)
    pad_idx = jnp.full((E_PAD - E_SL,), N, dtype=_i32)
    src = jnp.concatenate([edge_index[0], loop_idx, pad_idx])
    dst = jnp.concatenate([edge_index[1], loop_idx, pad_idx])

    x_pad = jnp.zeros((N_PAD, F), _f32).at[:N].set(x)

    dstg = dst >> 7
    dstc = dst & 127
    hoff = (jnp.arange(H1, dtype=_i32) * N_PAD)[:, None]
    srch = (src[None, :] + hoff).reshape(-1)
    dsth = (dst[None, :] + hoff).reshape(-1)

    xl1, xr1 = _tc_mm1(x_pad, Wl1, Wr1)
    msg1 = _sc_edge_layer(4)(xl1, xr1, srch, dsth, dst, dstg, dstc,
                             att1.reshape(-1))
    hl2, hr2 = _tc_mid(msg1, b1, Wl2, Wr2)
    num2, s2 = _sc_edge_layer(1)(hl2, hr2, src, dst, dst, dstg, dstc,
                                 att2.reshape(-1))
    out = _sc_merge()(num2, s2, b2)
    return out[:N]
